# baseline (speedup 1.0000x reference)
import functools

import jax
import jax.numpy as jnp
from jax import lax
from jax.experimental import pallas as pl
from jax.experimental.pallas import tpu as pltpu
from jax.experimental.pallas import tpu_sc as plsc

_N = 10000
_M = 32
_NPAD = 10240
_NW = 32
_NODES_PER_W = _NPAD // _NW
_GROUPS = _NODES_PER_W // 16
_CAPS = [8, 7, 6, 5, 4, 3]
_DIMS = [c * 8 for c in _CAPS]
_TCBLK = 2048


def _relu_linear_body(x_ref, w_ref, b_ref, o_ref):
    y = lax.dot_general(x_ref[...], w_ref[...], (((1,), (1,)), ((), ())),
                        preferred_element_type=jnp.float32)
    o_ref[...] = jnp.maximum(y + b_ref[...], 0.0)


def _tc_relu_linear(x, w, b2d):
    d_out, d_in = w.shape
    return pl.pallas_call(
        _relu_linear_body,
        grid=(_NPAD // _TCBLK,),
        in_specs=[
            pl.BlockSpec((_TCBLK, d_in), lambda i: (i, 0)),
            pl.BlockSpec((d_out, d_in), lambda i: (0, 0)),
            pl.BlockSpec((1, d_out), lambda i: (0, 0)),
        ],
        out_specs=pl.BlockSpec((_TCBLK, d_out), lambda i: (i, 0)),
        out_shape=jax.ShapeDtypeStruct((_NPAD, d_out), jnp.float32),
    )(x, w, b2d)


def _linear_norm_body(x_ref, w_ref, b_ref, e_ref, o_ref):
    y = lax.dot_general(x_ref[...], w_ref[...], (((1,), (1,)), ((), ())),
                        preferred_element_type=jnp.float32)
    y = y + b_ref[...]
    s = lax.dot_general(y * y, e_ref[...], (((1,), (1,)), ((), ())),
                        preferred_element_type=jnp.float32)
    inv = lax.rsqrt(jnp.maximum(s, 1e-24))
    o_ref[...] = y * lax.dot_general(inv, e_ref[...], (((1,), (0,)), ((), ())),
                                     preferred_element_type=jnp.float32)


def _tc_linear_norm(x, w, b2d, e):
    d_out, d_in = w.shape
    k = e.shape[0]
    return pl.pallas_call(
        _linear_norm_body,
        grid=(_NPAD // _TCBLK,),
        in_specs=[
            pl.BlockSpec((_TCBLK, d_in), lambda i: (i, 0)),
            pl.BlockSpec((d_out, d_in), lambda i: (0, 0)),
            pl.BlockSpec((1, d_out), lambda i: (0, 0)),
            pl.BlockSpec((k, d_out), lambda i: (0, 0)),
        ],
        out_specs=pl.BlockSpec((_TCBLK, d_out), lambda i: (i, 0)),
        out_shape=jax.ShapeDtypeStruct((_NPAD, d_out), jnp.float32),
    )(x, w, b2d, e)


def _splat_i(v):
    return jnp.full((16,), v, jnp.int32)


def _rsqrt16(x):
    xi = plsc.bitcast(x, jnp.int32)
    y = plsc.bitcast(jnp.int32(0x5F3759DF) - lax.shift_right_logical(xi, 1),
                     jnp.float32)
    for _ in range(3):
        y = y * (1.5 - 0.5 * x * y * y)
    return y


def _make_sc_routing(d, k):
    mesh = plsc.VectorSubcoreMesh(core_axis_name="c", subcore_axis_name="s",
                                  num_cores=2, num_subcores=16)

    @functools.partial(
        pl.kernel,
        out_type=jax.ShapeDtypeStruct((_NPAD, d), jnp.float32),
        mesh=mesh,
        scratch_types=[
            pltpu.VMEM((512,), jnp.int32),
            pltpu.VMEM((512, d), jnp.float32),
            pltpu.VMEM((16, d), jnp.float32),
            pltpu.VMEM((d, 16), jnp.float32),
            pltpu.VMEM((32 * k, 16), jnp.float32),
            pltpu.VMEM((16, d), jnp.float32),
            pltpu.SemaphoreType.DMA,
        ],
        compiler_params=pltpu.CompilerParams(needs_layout_passes=False,
                                             use_tc_tiling_on_sc=False),
    )
    def sc_fn(xk_hbm, nbf_hbm, out_hbm, idx_v, z_v, xs_v, u_v, p_v, o_v, sem):
        wid = lax.axis_index("s") * 2 + lax.axis_index("c")
        iota = lax.iota(jnp.int32, 16)
        zrow0 = iota * 32

        def u_pass(get_pv):
            for kk in range(k):
                c0 = kk * 8

                def mbody(m, accs):
                    row = zrow0 + m
                    pv = get_pv(m, kk)
                    return tuple(
                        accs[j] + pv * plsc.load_gather(z_v, [row, _splat_i(c0 + j)])
                        for j in range(8))

                accs = lax.fori_loop(
                    0, 32, mbody,
                    tuple(jnp.zeros((16,), jnp.float32) for _ in range(8)),
                    unroll=2)
                a = [accs[j] + plsc.load_gather(xs_v, [iota, _splat_i(c0 + j)])
                     for j in range(8)]
                n2 = a[0] * a[0]
                for j in range(1, 8):
                    n2 = n2 + a[j] * a[j]
                inv = _rsqrt16(jnp.maximum(n2, 1e-24))
                for j in range(8):
                    u_v[c0 + j] = a[j] * inv

        def do_group(g, carry):
            base = wid * _NODES_PER_W + g * 16
            pltpu.sync_copy(nbf_hbm.at[pl.ds(base * 32, 512)], idx_v)
            for q in range(4):
                pltpu.async_copy(xk_hbm.at[idx_v.at[pl.ds(q * 128, 128)]],
                                 z_v.at[pl.ds(q * 128, 128)], sem).wait()
            pltpu.sync_copy(xk_hbm.at[pl.ds(base, 16)], xs_v)

            invk = jnp.full((16,), 1.0 / k, jnp.float32)
            u_pass(lambda m, kk: invk)

            def iter_body(it, c2):
                for kk in range(k):
                    c0 = kk * 8
                    ukj = [u_v[c0 + j] for j in range(8)]

                    def pbody(m, c3):
                        row = zrow0 + m
                        s = ukj[0] * plsc.load_gather(z_v, [row, _splat_i(c0)])
                        for j in range(1, 8):
                            s = s + ukj[j] * plsc.load_gather(
                                z_v, [row, _splat_i(c0 + j)])
                        p_v[m * k + kk] = s
                        return c3

                    lax.fori_loop(0, 32, pbody, 0, unroll=2)

                def smbody(m, c3):
                    pv = [p_v[m * k + kk] for kk in range(k)]
                    mx = pv[0]
                    for kk in range(1, k):
                        mx = jnp.maximum(mx, pv[kk])
                    ev = [jnp.exp(t - mx) for t in pv]
                    ssum = ev[0]
                    for kk in range(1, k):
                        ssum = ssum + ev[kk]
                    inv = 1.0 / ssum
                    for kk in range(k):
                        p_v[m * k + kk] = ev[kk] * inv
                    return c3

                lax.fori_loop(0, 32, smbody, 0)

                u_pass(lambda m, kk: p_v[m * k + kk])
                return c2

            lax.fori_loop(0, 4, iter_body, 0)

            for e in range(d):
                plsc.store_scatter(o_v, [iota, _splat_i(e)],
                                   jnp.maximum(u_v[e], 0.0))
            pltpu.sync_copy(o_v, out_hbm.at[pl.ds(base, 16)])
            return carry

        lax.fori_loop(0, _GROUPS, do_group, 0)

    return sc_fn


_SC_ROUTING = [_make_sc_routing(_DIMS[i], _CAPS[i]) for i in range(6)]


def kernel(feature, neighbor_id, W_pca, b_pca, W0, b0, W1, b1, W2, b2,
           W3, b3, W4, b4, W5, b5):
    Ws = [W0, W1, W2, W3, W4, W5]
    bs = [b0, b1, b2, b3, b4, b5]
    featp = jnp.pad(feature, ((0, _NPAD - _N), (0, 0)))
    nbf = jnp.pad(neighbor_id, ((0, _NPAD - _N), (0, 0))).reshape(-1)

    x = _tc_relu_linear(featp, W_pca, b_pca.reshape(1, -1))
    xs_out = [x]
    for i in range(6):
        k, d = _CAPS[i], _DIMS[i]
        e = jnp.repeat(jnp.eye(k, dtype=jnp.float32), 8, axis=1)
        xk = _tc_linear_norm(x, Ws[i], bs[i].reshape(1, -1), e)
        x = _SC_ROUTING[i](xk, nbf)
        xs_out.append(x)
    return jnp.concatenate([t[:_N] for t in xs_out], axis=1)
